# baseline (speedup 1.0000x reference)
import functools

import jax
import jax.numpy as jnp
from jax import lax
from jax.experimental import pallas as pl
from jax.experimental.pallas import tpu as pltpu
from jax.experimental.pallas import tpu_sc as plsc

_HIDDEN = 768
_B, _L = 4096, 20
_LP = 32
_LG = 24

_NC, _NS, _LANES = 2, 16, 16
_NW = _NC * _NS
_ROWS_W = _B // _NW

_mesh = plsc.VectorSubcoreMesh(core_axis_name="c", subcore_axis_name="s")


@functools.partial(
    pl.kernel,
    mesh=_mesh,
    out_type=jax.ShapeDtypeStruct((_B, _L, _HIDDEN), jnp.float32),
    scratch_types=[
        pltpu.VMEM((_ROWS_W * _LP,), jnp.int32),
        pltpu.VMEM((_ROWS_W * _LP,), jnp.int32),
        pltpu.VMEM((_LG, _HIDDEN), jnp.float32),
        pltpu.VMEM((_LG, _HIDDEN), jnp.float32),
        pltpu.VMEM((_L, _HIDDEN), jnp.float32),
        pltpu.VMEM((_LG, _HIDDEN), jnp.float32),
        pltpu.VMEM((_LG, _HIDDEN), jnp.float32),
        pltpu.VMEM((_L, _HIDDEN), jnp.float32),
        pltpu.SemaphoreType.DMA,
        pltpu.SemaphoreType.DMA,
        pltpu.SemaphoreType.DMA,
        pltpu.SemaphoreType.DMA,
        pltpu.SemaphoreType.DMA,
        pltpu.SemaphoreType.DMA,
    ],
)
def _embed_sum(gsn_ids_hbm, route_ids_hbm, gsn_hbm, route_hbm, out_hbm,
               gidx, ridx, gbuf0, rbuf0, obuf0, gbuf1, rbuf1, obuf1,
               sem_g0, sem_g1, sem_r0, sem_r1, sem_o0, sem_o1):
    wid = lax.axis_index("s") * _NC + lax.axis_index("c")
    row_base = wid * _ROWS_W

    pltpu.sync_copy(gsn_ids_hbm.at[wid], gidx)
    pltpu.sync_copy(route_ids_hbm.at[wid], ridx)

    def add_rows(gbuf, rbuf, obuf):
        def row_body(i, c):
            for j in range(_HIDDEN // _LANES):
                sl = pl.ds(j * _LANES, _LANES)
                obuf[i, sl] = gbuf[i, sl] + rbuf[i, sl]
            return c
        lax.fori_loop(0, _L, row_body, 0)

    def start_gathers(b, gbuf, rbuf, sg, sr):
        isl = pl.ds(b * _LP, _LG)
        dg = pltpu.async_copy(gsn_hbm.at[gidx.at[isl]], gbuf, sg)
        dr = pltpu.async_copy(route_hbm.at[ridx.at[isl]], rbuf, sr)
        return dg, dr

    def start_writeback(b, obuf, so):
        return pltpu.async_copy(obuf, out_hbm.at[row_base + b], so)

    def group_body(g, carry):
        b0 = 2 * g
        b1 = b0 + 1
        dg0, dr0 = start_gathers(b0, gbuf0, rbuf0, sem_g0, sem_r0)
        dg1, dr1 = start_gathers(b1, gbuf1, rbuf1, sem_g1, sem_r1)

        dg0.wait()
        dr0.wait()
        add_rows(gbuf0, rbuf0, obuf0)
        wb0 = start_writeback(b0, obuf0, sem_o0)

        dg1.wait()
        dr1.wait()
        add_rows(gbuf1, rbuf1, obuf1)
        wb1 = start_writeback(b1, obuf1, sem_o1)

        wb0.wait()
        wb1.wait()
        return carry

    lax.fori_loop(0, _ROWS_W // 2, group_body, 0)


def _pad_ids(ids):
    ids = ids.astype(jnp.int32)
    ids = jnp.pad(ids, ((0, 0), (0, _LP - _L)))
    return ids.reshape(_NW, _ROWS_W * _LP)


def kernel(gsn_ids, route_ids, gsn_table, route_table):
    return _embed_sum(_pad_ids(gsn_ids), _pad_ids(route_ids),
                      gsn_table, route_table)
